# baseline (speedup 1.0000x reference)
import jax
import jax.numpy as jnp
from jax.experimental import pallas as pl
from jax.experimental.pallas import tpu as pltpu


def _moe_body(hs_ref, gate_ref, gu_ref, dn_ref, out_ref, w_ref):
    j = pl.program_id(0)
    t = pl.program_id(1)
    TC = hs_ref.shape[0]
    T, E = w_ref.shape
    F = dn_ref.shape[1]
    sl = pl.ds(t * TC, TC)

    @pl.when(j == 0)
    def _router():
        x = hs_ref[...]
        logits = jnp.dot(x, gate_ref[...], preferred_element_type=jnp.float32)
        p = jax.nn.softmax(logits, axis=-1)
        idx = jax.lax.broadcasted_iota(jnp.int32, p.shape, 1)
        m1 = jnp.max(p, axis=1, keepdims=True)
        i1 = jnp.min(jnp.where(p == m1, idx, E), axis=1, keepdims=True)
        sel1 = idx == i1
        p2 = jnp.where(sel1, -jnp.inf, p)
        m2 = jnp.max(p2, axis=1, keepdims=True)
        i2 = jnp.min(jnp.where(p2 == m2, idx, E), axis=1, keepdims=True)
        sel2 = idx == i2
        wsum = m1 + m2
        w = jnp.where(sel1, m1, jnp.where(sel2, m2, 0.0)) / wsum
        w_ref[sl, :] = w
        out_ref[sl, :] = jnp.zeros((TC, out_ref.shape[1]), jnp.float32)

    x = hs_ref[...].astype(jnp.bfloat16)
    lane = jax.lax.broadcasted_iota(jnp.int32, (TC, E), 1)
    wv = w_ref[sl, :]

    def expert(k):
        wgu = gu_ref[k].astype(jnp.bfloat16)
        gu = jnp.dot(x, wgu, preferred_element_type=jnp.float32)
        g = gu[:, :F]
        u = gu[:, F:]
        act = (u * (g * jax.nn.sigmoid(g))).astype(jnp.bfloat16)
        wdn = dn_ref[k].astype(jnp.bfloat16)
        d = jnp.dot(act, wdn, preferred_element_type=jnp.float32)
        e = 2 * j + k
        w_col = jnp.sum(jnp.where(lane == e, wv, 0.0), axis=1, keepdims=True)
        return w_col * d

    out_ref[sl, :] += expert(0) + expert(1)


def kernel(hidden_states, gate, gate_up_proj, down_proj):
    B, S, D = hidden_states.shape
    E, _, F2 = gate_up_proj.shape
    F = F2 // 2
    hs = hidden_states.reshape(-1, D)
    T = hs.shape[0]
    TCH = 8
    TC = T // TCH

    out = pl.pallas_call(
        _moe_body,
        grid=(E // 2, TCH),
        in_specs=[
            pl.BlockSpec((TC, D), lambda j, t: (t, 0)),
            pl.BlockSpec((D, E), lambda j, t: (0, 0)),
            pl.BlockSpec((2, D, F2), lambda j, t: (j, 0, 0)),
            pl.BlockSpec((2, F, D), lambda j, t: (j, 0, 0)),
        ],
        out_specs=pl.BlockSpec((T, D), lambda j, t: (0, 0)),
        out_shape=jax.ShapeDtypeStruct((T, D), jnp.float32),
        scratch_shapes=[pltpu.VMEM((T, E), jnp.float32)],
        compiler_params=pltpu.CompilerParams(
            dimension_semantics=("arbitrary", "arbitrary"),
            vmem_limit_bytes=100 * 1024 * 1024,
        ),
    )(hs, gate, gate_up_proj, down_proj)
    return out.reshape(B, S, D)


# speedup vs baseline: 1.2021x; 1.2021x over previous
import jax
import jax.numpy as jnp
from jax.experimental import pallas as pl
from jax.experimental.pallas import tpu as pltpu


def _moe_body(hs_ref, gate_ref, gu_ref, dn_ref, out_ref, w_ref):
    e = pl.program_id(0)
    T, E = w_ref.shape
    F = dn_ref.shape[1]

    @pl.when(e == 0)
    def _router():
        x = hs_ref[...]
        logits = jnp.dot(x, gate_ref[...], preferred_element_type=jnp.float32)
        p = jax.nn.softmax(logits, axis=-1)
        idx = jax.lax.broadcasted_iota(jnp.int32, p.shape, 1)
        m1 = jnp.max(p, axis=1, keepdims=True)
        i1 = jnp.min(jnp.where(p == m1, idx, E), axis=1, keepdims=True)
        sel1 = idx == i1
        p2 = jnp.where(sel1, -jnp.inf, p)
        m2 = jnp.max(p2, axis=1, keepdims=True)
        i2 = jnp.min(jnp.where(p2 == m2, idx, E), axis=1, keepdims=True)
        sel2 = idx == i2
        wsum = m1 + m2
        w = jnp.where(sel1, m1, jnp.where(sel2, m2, 0.0)) / wsum
        w_ref[...] = w
        out_ref[...] = jnp.zeros_like(out_ref)

    x = hs_ref[...].astype(jnp.bfloat16)
    wgu = gu_ref[0].astype(jnp.bfloat16)
    gu = jnp.dot(x, wgu, preferred_element_type=jnp.float32).astype(jnp.bfloat16)
    g = gu[:, :F]
    u = gu[:, F:]
    act = u * (g * jax.nn.sigmoid(g))
    wdn = dn_ref[0].astype(jnp.bfloat16)
    d = jnp.dot(act, wdn, preferred_element_type=jnp.float32)
    lane = jax.lax.broadcasted_iota(jnp.int32, (T, E), 1)
    w_col = jnp.sum(jnp.where(lane == e, w_ref[...], 0.0), axis=1, keepdims=True)
    out_ref[...] += w_col * d


def kernel(hidden_states, gate, gate_up_proj, down_proj):
    B, S, D = hidden_states.shape
    E, _, F2 = gate_up_proj.shape
    F = F2 // 2
    hs = hidden_states.reshape(-1, D)
    T = hs.shape[0]

    out = pl.pallas_call(
        _moe_body,
        grid=(E,),
        in_specs=[
            pl.BlockSpec((T, D), lambda e: (0, 0)),
            pl.BlockSpec((D, E), lambda e: (0, 0)),
            pl.BlockSpec((1, D, F2), lambda e: (e, 0, 0)),
            pl.BlockSpec((1, F, D), lambda e: (e, 0, 0)),
        ],
        out_specs=pl.BlockSpec((T, D), lambda e: (0, 0)),
        out_shape=jax.ShapeDtypeStruct((T, D), jnp.float32),
        scratch_shapes=[pltpu.VMEM((T, E), jnp.float32)],
        compiler_params=pltpu.CompilerParams(
            dimension_semantics=("arbitrary",),
        ),
    )(hs, gate, gate_up_proj, down_proj)
    return out.reshape(B, S, D)


# speedup vs baseline: 1.2174x; 1.0127x over previous
import jax
import jax.numpy as jnp
from jax.experimental import pallas as pl
from jax.experimental.pallas import tpu as pltpu


def _moe_body(hs_ref, gate_ref, gu_ref, dn_ref, out_ref, w_ref):
    e = pl.program_id(0)
    T, E = w_ref.shape
    F = dn_ref.shape[1]

    @pl.when(e == 0)
    def _router():
        x = hs_ref[...]
        logits = jnp.dot(x, gate_ref[...], preferred_element_type=jnp.float32)
        p = jax.nn.softmax(logits, axis=-1)
        idx = jax.lax.broadcasted_iota(jnp.int32, p.shape, 1)
        m1 = jnp.max(p, axis=1, keepdims=True)
        i1 = jnp.min(jnp.where(p == m1, idx, E), axis=1, keepdims=True)
        sel1 = idx == i1
        p2 = jnp.where(sel1, -jnp.inf, p)
        m2 = jnp.max(p2, axis=1, keepdims=True)
        i2 = jnp.min(jnp.where(p2 == m2, idx, E), axis=1, keepdims=True)
        sel2 = idx == i2
        wsum = m1 + m2
        w = jnp.where(sel1, m1, jnp.where(sel2, m2, 0.0)) / wsum
        w_ref[...] = w
        out_ref[...] = jnp.zeros_like(out_ref)

    x = hs_ref[...]
    gu = jnp.dot(x, gu_ref[0], preferred_element_type=jnp.float32)
    g = gu[:, :F]
    u = gu[:, F:]
    act = u * (g * jax.nn.sigmoid(g))
    d = jnp.dot(act, dn_ref[0], preferred_element_type=jnp.float32)
    lane = jax.lax.broadcasted_iota(jnp.int32, (T, E), 1)
    w_col = jnp.sum(jnp.where(lane == e, w_ref[...], 0.0), axis=1, keepdims=True)
    out_ref[...] += w_col * d


def kernel(hidden_states, gate, gate_up_proj, down_proj):
    B, S, D = hidden_states.shape
    E, _, F2 = gate_up_proj.shape
    F = F2 // 2
    hs = hidden_states.reshape(-1, D)
    T = hs.shape[0]

    out = pl.pallas_call(
        _moe_body,
        grid=(E,),
        in_specs=[
            pl.BlockSpec((T, D), lambda e: (0, 0)),
            pl.BlockSpec((D, E), lambda e: (0, 0)),
            pl.BlockSpec((1, D, F2), lambda e: (e, 0, 0)),
            pl.BlockSpec((1, F, D), lambda e: (e, 0, 0)),
        ],
        out_specs=pl.BlockSpec((T, D), lambda e: (0, 0)),
        out_shape=jax.ShapeDtypeStruct((T, D), jnp.float32),
        scratch_shapes=[pltpu.VMEM((T, E), jnp.float32)],
        compiler_params=pltpu.CompilerParams(
            dimension_semantics=("arbitrary",),
        ),
    )(hs, gate, gate_up_proj, down_proj)
    return out.reshape(B, S, D)
